# baseline (speedup 1.0000x reference)
import functools

import jax
import jax.numpy as jnp
from jax import lax
from jax.experimental import pallas as pl
from jax.experimental.pallas import tpu as pltpu
from jax.experimental.pallas import tpu_sc as plsc


def _sc_gather_pair(h, src, dst):
    N, D = h.shape
    E = src.shape[0]
    NW = 32
    per_w = E // NW
    K = 200
    nch = per_w // K
    dt = h.dtype
    mesh = plsc.VectorSubcoreMesh(core_axis_name="c", subcore_axis_name="s")

    @functools.partial(
        pl.kernel, mesh=mesh,
        out_type=[jax.ShapeDtypeStruct((E, D), dt),
                  jax.ShapeDtypeStruct((E, D), dt)],
        scratch_types=[pltpu.VMEM((K,), jnp.int32),
                       pltpu.VMEM((K, D), dt),
                       pltpu.SemaphoreType.DMA],
    )
    def gk(h_hbm, src_hbm, dst_hbm, hs_hbm, hd_hbm, idx_v, rows_v, sem):
        wid = lax.axis_index("s") * 2 + lax.axis_index("c")
        base = wid * per_w
        for j in range(nch):
            off = base + j * K
            pltpu.sync_copy(src_hbm.at[pl.ds(off, K)], idx_v)
            pltpu.async_copy(h_hbm.at[idx_v], rows_v, sem).wait()
            pltpu.sync_copy(rows_v, hs_hbm.at[pl.ds(off, K), :])
            pltpu.sync_copy(dst_hbm.at[pl.ds(off, K)], idx_v)
            pltpu.async_copy(h_hbm.at[idx_v], rows_v, sem).wait()
            pltpu.sync_copy(rows_v, hd_hbm.at[pl.ds(off, K), :])

    return gk(h, src, dst)


def _unpack_bf16_pair(xi):
    f32 = jnp.float32
    lo = jax.lax.bitcast_convert_type(xi << 16, f32)
    hi = jax.lax.bitcast_convert_type(xi & jnp.int32(-65536), f32)
    return lo.astype(jnp.bfloat16), hi.astype(jnp.bfloat16)


def _edge_mlp_body(hs_ref, hd_ref, ed_ref,
                   Wab_e_ref, Wab_o_ref, Wbb_e_ref, Wbb_o_ref,
                   Web_ref, bf_ref,
                   W2_ref, b2_ref, wc2_ref,
                   We1_ref, be1_ref, We2_ref, be2_ref,
                   m_ref, cw_ref, *, H):
    f32 = jnp.float32
    bf16 = jnp.bfloat16
    d = ed_ref[...]
    eh = d * We1_ref[...] + be1_ref[...]
    eh = eh * jax.nn.sigmoid(eh)
    ea = jnp.dot(eh, We2_ref[...], preferred_element_type=f32) + be2_ref[...]
    hsA, hsB = _unpack_bf16_pair(hs_ref[...])
    hdA, hdB = _unpack_bf16_pair(hd_ref[...])
    pre = (jnp.dot(hsA, Wab_e_ref[...], preferred_element_type=f32)
           + jnp.dot(hsB, Wab_o_ref[...], preferred_element_type=f32)
           + jnp.dot(hdA, Wbb_e_ref[...], preferred_element_type=f32)
           + jnp.dot(hdB, Wbb_o_ref[...], preferred_element_type=f32)
           + jnp.dot(ea.astype(bf16), Web_ref[...], preferred_element_type=f32)
           + bf_ref[...])
    preb = pre.astype(bf16)
    act = preb * jax.nn.sigmoid(preb)
    a_node = act[:, :H]
    a_coord = act[:, H:]
    m_ref[...] = jnp.dot(a_node, W2_ref[...], preferred_element_type=f32) + b2_ref[...]
    cw_ref[...] = jnp.sum(a_coord * wc2_ref[...], axis=1)


def _sc_scatter(h, x0, x1, x2, pxs, m, cw, src, dst):
    N, D = h.shape
    E = src.shape[0]
    HC = D // 2
    K = 200
    nh = E // 16 // K
    KX = 320
    ncx = E // 2 // KX
    rows_t = (N // 16) & ~7
    tail0 = rows_t * 16
    ntail = N - tail0
    mesh = plsc.VectorSubcoreMesh(core_axis_name="c", subcore_axis_name="s")

    @functools.partial(
        pl.kernel, mesh=mesh,
        out_type=[jax.ShapeDtypeStruct((N, D), jnp.float32)]
                 + [jax.ShapeDtypeStruct((N,), jnp.float32)] * 6,
        scratch_types=[pltpu.VMEM_SHARED((N, HC), jnp.float32),
                       pltpu.VMEM_SHARED((N,), jnp.float32),
                       pltpu.VMEM_SHARED((N,), jnp.float32),
                       pltpu.VMEM_SHARED((N,), jnp.float32),
                       pltpu.VMEM((K,), jnp.int32),
                       pltpu.VMEM((K, HC), jnp.float32),
                       pltpu.VMEM((KX,), jnp.int32),
                       pltpu.VMEM((KX,), jnp.int32),
                       pltpu.VMEM((KX,), jnp.float32),
                       pltpu.VMEM((KX,), jnp.float32),
                       pltpu.VMEM((KX,), jnp.float32),
                       pltpu.VMEM((KX,), jnp.float32),
                       pltpu.SemaphoreType.DMA],
    )
    def sk(h_hbm, x0_hbm, x1_hbm, x2_hbm,
           i00, i01, i02, i10, i11, i12, m_hbm, cw_hbm,
           src_hbm, dst_hbm, ho_hbm, p00, p01, p02, p10, p11, p12,
           hacc, xa0, xa1, xa2, dstv, mv, srcv, dstxv, cwv, xsv, xdv, cuv,
           sem):
        c = lax.axis_index("c")
        s = lax.axis_index("s")
        r0 = pl.multiple_of(s * rows_t, 8)
        col = pl.multiple_of(c * HC, HC)
        xaccs = [xa0, xa1, xa2]
        xcols = [x0_hbm, x1_hbm, x2_hbm]
        pltpu.sync_copy(h_hbm.at[pl.ds(r0, rows_t), pl.ds(col, HC)],
                        hacc.at[pl.ds(r0, rows_t), :])
        if ntail:
            @pl.when(s == 0)
            def _():
                pltpu.sync_copy(h_hbm.at[pl.ds(tail0, ntail), pl.ds(col, HC)],
                                hacc.at[pl.ds(tail0, ntail), :])
        @pl.when((s == 0) & (c == 0))
        def _():
            for q, src_ref in enumerate([i00, i01, i02]):
                pltpu.sync_copy(src_ref, xaccs[q])
        @pl.when((s == 0) & (c == 1))
        def _():
            for q, src_ref in enumerate([i10, i11, i12]):
                pltpu.sync_copy(src_ref, xaccs[q])
        plsc.subcore_barrier()

        hbase = s * (E // 16)
        def hchunk(j, _):
            off = pl.multiple_of(hbase + j * K, 8)
            pltpu.sync_copy(dst_hbm.at[pl.ds(off, K)], dstv)
            pltpu.sync_copy(m_hbm.at[pl.ds(off, K), pl.ds(col, HC)], mv)
            pltpu.sync_copy(mv, hacc.at[dstv], add=True)
            return _
        lax.fori_loop(0, nh, hchunk, 0)

        nj = (ncx - s + 15) // 16
        def xchunk(j, _):
            cidx = s + j * 16
            off = pl.multiple_of(c * (E // 2) + cidx * KX, 8)
            pltpu.sync_copy(src_hbm.at[pl.ds(off, KX)], srcv)
            pltpu.sync_copy(dst_hbm.at[pl.ds(off, KX)], dstxv)
            pltpu.sync_copy(cw_hbm.at[pl.ds(off, KX)], cwv)
            for q in range(3):
                pltpu.async_copy(xcols[q].at[srcv], xsv, sem).wait()
                pltpu.async_copy(xcols[q].at[dstxv], xdv, sem).wait()
                for g in range(KX // 16):
                    d16 = pl.ds(g * 16, 16)
                    cuv[d16] = cwv[d16] * (xsv[d16] - xdv[d16])
                pltpu.sync_copy(cuv, xaccs[q].at[dstxv], add=True)
            return _
        lax.fori_loop(0, nj, xchunk, 0)
        plsc.subcore_barrier()

        pltpu.sync_copy(hacc.at[pl.ds(r0, rows_t), :],
                        ho_hbm.at[pl.ds(r0, rows_t), pl.ds(col, HC)])
        if ntail:
            @pl.when(s == 0)
            def _():
                pltpu.sync_copy(hacc.at[pl.ds(tail0, ntail), :],
                                ho_hbm.at[pl.ds(tail0, ntail), pl.ds(col, HC)])
        @pl.when((s == 0) & (c == 0))
        def _():
            for q, dst_ref in enumerate([p00, p01, p02]):
                pltpu.sync_copy(xaccs[q], dst_ref)
        @pl.when((s == 0) & (c == 1))
        def _():
            for q, dst_ref in enumerate([p10, p11, p12]):
                pltpu.sync_copy(xaccs[q], dst_ref)

    return sk(h, x0, x1, x2, *pxs, m, cw, src, dst)


def _pick_block(E):
    for b in (256, 128, 64, 32, 16, 8):
        if E % b == 0:
            return b
    return 8


def kernel(h, x, edge_index, edge_dist, W1, b1, W2, b2, Wc1, bc1, Wc2, We1, be1, We2, be2):
    N, D = h.shape
    E = edge_dist.shape[0]
    H = W1.shape[1]
    ED = We2.shape[0]
    B = _pick_block(E)
    src = edge_index[0]
    dst = edge_index[1]

    bf16 = jnp.bfloat16
    Wab = jnp.concatenate([W1[:D], Wc1[:D]], axis=1).astype(bf16)
    Wbb = jnp.concatenate([W1[D:2 * D], Wc1[D:2 * D]], axis=1).astype(bf16)
    Web = jnp.concatenate([W1[2 * D:], Wc1[2 * D:]], axis=1).astype(bf16)
    bf = jnp.concatenate([b1, bc1])[None, :]
    b2r = b2[None, :]
    wc2r = Wc2[:, 0][None, :]
    be1r = be1[None, :]
    be2r = be2[None, :]

    weights = (Wab[0::2], Wab[1::2], Wbb[0::2], Wbb[1::2],
               Web, bf, W2.astype(bf16), b2r, wc2r,
               We1, be1r, We2, be2r)
    full = lambda r, c: pl.BlockSpec((r, c), lambda i: (0, 0))

    def edge_mlp(hs, hd, ed2):
        ES = hs.shape[0]
        return pl.pallas_call(
            functools.partial(_edge_mlp_body, H=H),
            grid=(ES // B,),
            in_specs=[
                pl.BlockSpec((B, D // 2), lambda i: (i, 0)),
                pl.BlockSpec((B, D // 2), lambda i: (i, 0)),
                pl.BlockSpec((B, 1), lambda i: (i, 0)),
                full(D // 2, 2 * H), full(D // 2, 2 * H),
                full(D // 2, 2 * H), full(D // 2, 2 * H),
                full(ED, 2 * H), full(1, 2 * H),
                full(H, D), full(1, D), full(1, H),
                full(1, ED), full(1, ED), full(ED, ED), full(1, ED),
            ],
            out_specs=[
                pl.BlockSpec((B, D), lambda i: (i, 0)),
                pl.BlockSpec((B,), lambda i: (i,)),
            ],
            out_shape=[
                jax.ShapeDtypeStruct((ES, D), jnp.float32),
                jax.ShapeDtypeStruct((ES,), jnp.float32),
            ],
        )(hs, hd, ed2, *weights)

    S = 5 if E % (5 * 32 * 200) == 0 else 1
    ES = E // S
    ho = h
    hb = jax.lax.bitcast_convert_type(
        h.astype(bf16).reshape(N, D // 2, 2), jnp.int32)
    pxs = [jnp.zeros((N,), jnp.float32)] * 6
    x0, x1, x2 = x[:, 0], x[:, 1], x[:, 2]
    for si in range(S):
        sl = slice(si * ES, (si + 1) * ES)
        hs, hd = _sc_gather_pair(hb, src[sl], dst[sl])
        m, cw = edge_mlp(hs, hd, edge_dist[sl, None])
        ho, *pxs = _sc_scatter(ho, x0, x1, x2, pxs, m, cw, src[sl], dst[sl])
    p00, p01, p02, p10, p11, p12 = pxs
    x_out = x + jnp.stack([p00 + p10, p01 + p11, p02 + p12], axis=1)
    return (ho, x_out)


# speedup vs baseline: 1.0100x; 1.0100x over previous
import functools

import jax
import jax.numpy as jnp
from jax import lax
from jax.experimental import pallas as pl
from jax.experimental.pallas import tpu as pltpu
from jax.experimental.pallas import tpu_sc as plsc


def _sc_gather_pair(h, src, dst):
    N, D = h.shape
    E = src.shape[0]
    NW = 32
    per_w = E // NW
    K = 200
    nch = per_w // K
    dt = h.dtype
    mesh = plsc.VectorSubcoreMesh(core_axis_name="c", subcore_axis_name="s")

    @functools.partial(
        pl.kernel, mesh=mesh,
        out_type=[jax.ShapeDtypeStruct((E, D), dt),
                  jax.ShapeDtypeStruct((E, D), dt)],
        scratch_types=[pltpu.VMEM((K,), jnp.int32),
                       pltpu.VMEM((K, D), dt),
                       pltpu.SemaphoreType.DMA],
    )
    def gk(h_hbm, src_hbm, dst_hbm, hs_hbm, hd_hbm, idx_v, rows_v, sem):
        wid = lax.axis_index("s") * 2 + lax.axis_index("c")
        base = wid * per_w
        for j in range(nch):
            off = base + j * K
            pltpu.sync_copy(src_hbm.at[pl.ds(off, K)], idx_v)
            pltpu.async_copy(h_hbm.at[idx_v], rows_v, sem).wait()
            pltpu.sync_copy(rows_v, hs_hbm.at[pl.ds(off, K), :])
            pltpu.sync_copy(dst_hbm.at[pl.ds(off, K)], idx_v)
            pltpu.async_copy(h_hbm.at[idx_v], rows_v, sem).wait()
            pltpu.sync_copy(rows_v, hd_hbm.at[pl.ds(off, K), :])

    return gk(h, src, dst)


def _unpack_bf16_pair(xi):
    f32 = jnp.float32
    lo = jax.lax.bitcast_convert_type(xi << 16, f32)
    hi = jax.lax.bitcast_convert_type(xi & jnp.int32(-65536), f32)
    return lo.astype(jnp.bfloat16), hi.astype(jnp.bfloat16)


def _edge_mlp_body(hs_ref, hd_ref, ed_ref,
                   Wab_e_ref, Wab_o_ref, Wbb_e_ref, Wbb_o_ref,
                   Web_ref, bf_ref,
                   W2_ref, b2_ref, wc2_ref,
                   We1_ref, be1_ref, We2_ref, be2_ref,
                   m_ref, cw_ref, *, H):
    f32 = jnp.float32
    bf16 = jnp.bfloat16
    d = ed_ref[...]
    eh = d * We1_ref[...] + be1_ref[...]
    eh = eh * jax.nn.sigmoid(eh)
    ea = jnp.dot(eh, We2_ref[...], preferred_element_type=f32) + be2_ref[...]
    hsA, hsB = _unpack_bf16_pair(hs_ref[...])
    hdA, hdB = _unpack_bf16_pair(hd_ref[...])
    pre = (jnp.dot(hsA, Wab_e_ref[...], preferred_element_type=f32)
           + jnp.dot(hsB, Wab_o_ref[...], preferred_element_type=f32)
           + jnp.dot(hdA, Wbb_e_ref[...], preferred_element_type=f32)
           + jnp.dot(hdB, Wbb_o_ref[...], preferred_element_type=f32)
           + jnp.dot(ea.astype(bf16), Web_ref[...], preferred_element_type=f32)
           + bf_ref[...])
    act = ((pre * 0.5) * (1.0 + jnp.tanh(pre * 0.5))).astype(bf16)
    a_node = act[:, :H]
    a_coord = act[:, H:]
    m_ref[...] = jnp.dot(a_node, W2_ref[...], preferred_element_type=f32) + b2_ref[...]
    cw_ref[...] = jnp.dot(a_coord, wc2_ref[...], preferred_element_type=f32)


def _sc_scatter(h, x0, x1, x2, pxs, m, cw, src, dst):
    N, D = h.shape
    E = src.shape[0]
    HC = D // 2
    K = 200
    nh = E // 16 // K
    KX = 320
    ncx = E // 2 // KX
    rows_t = (N // 16) & ~7
    tail0 = rows_t * 16
    ntail = N - tail0
    mesh = plsc.VectorSubcoreMesh(core_axis_name="c", subcore_axis_name="s")

    @functools.partial(
        pl.kernel, mesh=mesh,
        out_type=[jax.ShapeDtypeStruct((N, D), jnp.float32)]
                 + [jax.ShapeDtypeStruct((N,), jnp.float32)] * 6,
        scratch_types=[pltpu.VMEM_SHARED((N, HC), jnp.float32),
                       pltpu.VMEM_SHARED((N,), jnp.float32),
                       pltpu.VMEM_SHARED((N,), jnp.float32),
                       pltpu.VMEM_SHARED((N,), jnp.float32),
                       pltpu.VMEM((K,), jnp.int32),
                       pltpu.VMEM((K, HC), jnp.float32),
                       pltpu.VMEM((KX,), jnp.int32),
                       pltpu.VMEM((KX,), jnp.int32),
                       pltpu.VMEM((KX,), jnp.float32),
                       pltpu.VMEM((KX,), jnp.float32),
                       pltpu.VMEM((KX,), jnp.float32),
                       pltpu.VMEM((KX,), jnp.float32),
                       pltpu.SemaphoreType.DMA],
    )
    def sk(h_hbm, x0_hbm, x1_hbm, x2_hbm,
           i00, i01, i02, i10, i11, i12, m_hbm, cw_hbm,
           src_hbm, dst_hbm, ho_hbm, p00, p01, p02, p10, p11, p12,
           hacc, xa0, xa1, xa2, dstv, mv, srcv, dstxv, cwv, xsv, xdv, cuv,
           sem):
        c = lax.axis_index("c")
        s = lax.axis_index("s")
        r0 = pl.multiple_of(s * rows_t, 8)
        col = pl.multiple_of(c * HC, HC)
        xaccs = [xa0, xa1, xa2]
        xcols = [x0_hbm, x1_hbm, x2_hbm]
        pltpu.sync_copy(h_hbm.at[pl.ds(r0, rows_t), pl.ds(col, HC)],
                        hacc.at[pl.ds(r0, rows_t), :])
        if ntail:
            @pl.when(s == 0)
            def _():
                pltpu.sync_copy(h_hbm.at[pl.ds(tail0, ntail), pl.ds(col, HC)],
                                hacc.at[pl.ds(tail0, ntail), :])
        @pl.when((s == 0) & (c == 0))
        def _():
            for q, src_ref in enumerate([i00, i01, i02]):
                pltpu.sync_copy(src_ref, xaccs[q])
        @pl.when((s == 0) & (c == 1))
        def _():
            for q, src_ref in enumerate([i10, i11, i12]):
                pltpu.sync_copy(src_ref, xaccs[q])
        plsc.subcore_barrier()

        hbase = s * (E // 16)
        def hchunk(j, _):
            off = pl.multiple_of(hbase + j * K, 8)
            pltpu.sync_copy(dst_hbm.at[pl.ds(off, K)], dstv)
            pltpu.sync_copy(m_hbm.at[pl.ds(off, K), pl.ds(col, HC)], mv)
            pltpu.sync_copy(mv, hacc.at[dstv], add=True)
            return _
        lax.fori_loop(0, nh, hchunk, 0)

        nj = (ncx - s + 15) // 16
        def xchunk(j, _):
            cidx = s + j * 16
            off = pl.multiple_of(c * (E // 2) + cidx * KX, 8)
            pltpu.sync_copy(src_hbm.at[pl.ds(off, KX)], srcv)
            pltpu.sync_copy(dst_hbm.at[pl.ds(off, KX)], dstxv)
            pltpu.sync_copy(cw_hbm.at[pl.ds(off, KX)], cwv)
            for q in range(3):
                pltpu.async_copy(xcols[q].at[srcv], xsv, sem).wait()
                pltpu.async_copy(xcols[q].at[dstxv], xdv, sem).wait()
                for g in range(KX // 16):
                    d16 = pl.ds(g * 16, 16)
                    cuv[d16] = cwv[d16] * (xsv[d16] - xdv[d16])
                pltpu.sync_copy(cuv, xaccs[q].at[dstxv], add=True)
            return _
        lax.fori_loop(0, nj, xchunk, 0)
        plsc.subcore_barrier()

        pltpu.sync_copy(hacc.at[pl.ds(r0, rows_t), :],
                        ho_hbm.at[pl.ds(r0, rows_t), pl.ds(col, HC)])
        if ntail:
            @pl.when(s == 0)
            def _():
                pltpu.sync_copy(hacc.at[pl.ds(tail0, ntail), :],
                                ho_hbm.at[pl.ds(tail0, ntail), pl.ds(col, HC)])
        @pl.when((s == 0) & (c == 0))
        def _():
            for q, dst_ref in enumerate([p00, p01, p02]):
                pltpu.sync_copy(xaccs[q], dst_ref)
        @pl.when((s == 0) & (c == 1))
        def _():
            for q, dst_ref in enumerate([p10, p11, p12]):
                pltpu.sync_copy(xaccs[q], dst_ref)

    return sk(h, x0, x1, x2, *pxs, m, cw, src, dst)


def _pick_block(E):
    for b in (256, 128, 64, 32, 16, 8):
        if E % b == 0:
            return b
    return 8


def kernel(h, x, edge_index, edge_dist, W1, b1, W2, b2, Wc1, bc1, Wc2, We1, be1, We2, be2):
    N, D = h.shape
    E = edge_dist.shape[0]
    H = W1.shape[1]
    ED = We2.shape[0]
    B = _pick_block(E)
    src = edge_index[0]
    dst = edge_index[1]

    bf16 = jnp.bfloat16
    Wab = jnp.concatenate([W1[:D], Wc1[:D]], axis=1).astype(bf16)
    Wbb = jnp.concatenate([W1[D:2 * D], Wc1[D:2 * D]], axis=1).astype(bf16)
    Web = jnp.concatenate([W1[2 * D:], Wc1[2 * D:]], axis=1).astype(bf16)
    bf = jnp.concatenate([b1, bc1])[None, :]
    b2r = b2[None, :]
    wc2r = Wc2[:, 0].astype(bf16)
    be1r = be1[None, :]
    be2r = be2[None, :]

    weights = (Wab[0::2], Wab[1::2], Wbb[0::2], Wbb[1::2],
               Web, bf, W2.astype(bf16), b2r, wc2r,
               We1, be1r, We2, be2r)
    full = lambda r, c: pl.BlockSpec((r, c), lambda i: (0, 0))

    def edge_mlp(hs, hd, ed2):
        ES = hs.shape[0]
        return pl.pallas_call(
            functools.partial(_edge_mlp_body, H=H),
            grid=(ES // B,),
            in_specs=[
                pl.BlockSpec((B, D // 2), lambda i: (i, 0)),
                pl.BlockSpec((B, D // 2), lambda i: (i, 0)),
                pl.BlockSpec((B, 1), lambda i: (i, 0)),
                full(D // 2, 2 * H), full(D // 2, 2 * H),
                full(D // 2, 2 * H), full(D // 2, 2 * H),
                full(ED, 2 * H), full(1, 2 * H),
                full(H, D), full(1, D),
                pl.BlockSpec((H,), lambda i: (0,)),
                full(1, ED), full(1, ED), full(ED, ED), full(1, ED),
            ],
            out_specs=[
                pl.BlockSpec((B, D), lambda i: (i, 0)),
                pl.BlockSpec((B,), lambda i: (i,)),
            ],
            out_shape=[
                jax.ShapeDtypeStruct((ES, D), jnp.float32),
                jax.ShapeDtypeStruct((ES,), jnp.float32),
            ],
        )(hs, hd, ed2, *weights)

    S = 5 if E % (5 * 32 * 200) == 0 else 1
    ES = E // S
    ho = h
    hb = jax.lax.bitcast_convert_type(
        h.astype(bf16).reshape(N, D // 2, 2), jnp.int32)
    pxs = [jnp.zeros((N,), jnp.float32)] * 6
    x0, x1, x2 = x[:, 0], x[:, 1], x[:, 2]
    for si in range(S):
        sl = slice(si * ES, (si + 1) * ES)
        hs, hd = _sc_gather_pair(hb, src[sl], dst[sl])
        m, cw = edge_mlp(hs, hd, edge_dist[sl, None])
        ho, *pxs = _sc_scatter(ho, x0, x1, x2, pxs, m, cw, src[sl], dst[sl])
    p00, p01, p02, p10, p11, p12 = pxs
    x_out = x + jnp.stack([p00 + p10, p01 + p11, p02 + p12], axis=1)
    return (ho, x_out)


# speedup vs baseline: 1.0744x; 1.0638x over previous
import functools

import jax
import jax.numpy as jnp
from jax import lax
from jax.experimental import pallas as pl
from jax.experimental.pallas import tpu as pltpu
from jax.experimental.pallas import tpu_sc as plsc


def _sc_gather_pair(h, src, dst):
    N, D = h.shape
    E = src.shape[0]
    NW = 32
    per_w = E // NW
    K = 200
    nch = per_w // K
    dt = h.dtype
    mesh = plsc.VectorSubcoreMesh(core_axis_name="c", subcore_axis_name="s")

    @functools.partial(
        pl.kernel, mesh=mesh,
        out_type=[jax.ShapeDtypeStruct((E, D), dt),
                  jax.ShapeDtypeStruct((E, D), dt)],
        scratch_types=[pltpu.VMEM((2 * per_w,), jnp.int32),
                       pltpu.VMEM((K, D), dt),
                       pltpu.VMEM((K, D), dt),
                       pltpu.SemaphoreType.DMA,
                       pltpu.SemaphoreType.DMA,
                       pltpu.SemaphoreType.DMA,
                       pltpu.SemaphoreType.DMA],
    )
    def gk(h_hbm, src_hbm, dst_hbm, hs_hbm, hd_hbm,
           idx_v, rows0, rows1, sg0, sg1, sw0, sw1):
        wid = lax.axis_index("s") * 2 + lax.axis_index("c")
        base = wid * per_w
        pltpu.sync_copy(src_hbm.at[pl.ds(base, per_w)],
                        idx_v.at[pl.ds(0, per_w)])
        pltpu.sync_copy(dst_hbm.at[pl.ds(base, per_w)],
                        idx_v.at[pl.ds(per_w, per_w)])
        rows = (rows0, rows1)
        sg = (sg0, sg1)
        sw = (sw0, sw1)
        chunks = []
        for t in range(2 * nch):
            out = hs_hbm if t < nch else hd_hbm
            ioff = t * K
            ooff = base + (t % nch) * K
            chunks.append((out, ioff, ooff))
        gathers = []
        writes = [None, None]
        for t, (out, ioff, ooff) in enumerate(chunks):
            b = t % 2
            if writes[b] is not None:
                writes[b].wait()
            g = pltpu.async_copy(h_hbm.at[idx_v.at[pl.ds(ioff, K)]],
                                 rows[b], sg[b])
            gathers.append((g, t))
            if t >= 1:
                gprev, tp = gathers[t - 1]
                gprev.wait()
                outp, _, ooffp = chunks[tp]
                bp = tp % 2
                writes[bp] = pltpu.async_copy(
                    rows[bp], outp.at[pl.ds(ooffp, K), :], sw[bp])
        glast, tl = gathers[-1]
        glast.wait()
        outl, _, ooffl = chunks[tl]
        pltpu.sync_copy(rows[tl % 2], outl.at[pl.ds(ooffl, K), :])
        if writes[(tl - 1) % 2] is not None:
            writes[(tl - 1) % 2].wait()

    return gk(h, src, dst)


def _edge_mlp_body(hs_ref, hd_ref, ed_ref,
                   Wab_ref, Wbb_ref, Web_ref, bf_ref,
                   W2_ref, b2_ref, wc2_ref,
                   We1_ref, be1_ref, We2_ref, be2_ref,
                   m_ref, cw_ref, *, H):
    f32 = jnp.float32
    bf16 = jnp.bfloat16
    pre = (jnp.dot(hs_ref[...].astype(bf16), Wab_ref[...], preferred_element_type=f32)
           + jnp.dot(hd_ref[...].astype(bf16), Wbb_ref[...], preferred_element_type=f32)
           + bf_ref[...])
    d = ed_ref[...]
    eh = d * We1_ref[...] + be1_ref[...]
    eh = eh * jax.nn.sigmoid(eh)
    ea = jnp.dot(eh, We2_ref[...], preferred_element_type=f32) + be2_ref[...]
    pre = pre + jnp.dot(ea.astype(bf16), Web_ref[...], preferred_element_type=f32)
    preb = pre.astype(bf16)
    act = preb * jax.nn.sigmoid(preb)
    a_node = act[:, :H]
    a_coord = act[:, H:]
    m_ref[...] = jnp.dot(a_node, W2_ref[...], preferred_element_type=f32) + b2_ref[...]
    cw_ref[...] = jnp.sum(a_coord * wc2_ref[...].astype(f32), axis=1)


def _sc_scatter(h, x0, x1, x2, pxs, m, cw, src, dst):
    N, D = h.shape
    E = src.shape[0]
    HC = D // 2
    K = 200
    nh = E // 16 // K
    KX = 320
    ncx = E // 2 // KX
    rows_t = (N // 16) & ~7
    tail0 = rows_t * 16
    ntail = N - tail0
    mesh = plsc.VectorSubcoreMesh(core_axis_name="c", subcore_axis_name="s")

    @functools.partial(
        pl.kernel, mesh=mesh,
        out_type=[jax.ShapeDtypeStruct((N, D), jnp.float32)]
                 + [jax.ShapeDtypeStruct((N,), jnp.float32)] * 6,
        scratch_types=[pltpu.VMEM_SHARED((N, HC), jnp.float32),
                       pltpu.VMEM_SHARED((N,), jnp.float32),
                       pltpu.VMEM_SHARED((N,), jnp.float32),
                       pltpu.VMEM_SHARED((N,), jnp.float32),
                       pltpu.VMEM((K,), jnp.int32),
                       pltpu.VMEM((K, HC), jnp.float32),
                       pltpu.VMEM((KX,), jnp.int32),
                       pltpu.VMEM((KX,), jnp.int32),
                       pltpu.VMEM((KX,), jnp.float32),
                       pltpu.VMEM((KX,), jnp.float32),
                       pltpu.VMEM((KX,), jnp.float32),
                       pltpu.VMEM((KX,), jnp.float32),
                       pltpu.SemaphoreType.DMA],
    )
    def sk(h_hbm, x0_hbm, x1_hbm, x2_hbm,
           i00, i01, i02, i10, i11, i12, m_hbm, cw_hbm,
           src_hbm, dst_hbm, ho_hbm, p00, p01, p02, p10, p11, p12,
           hacc, xa0, xa1, xa2, dstv, mv, srcv, dstxv, cwv, xsv, xdv, cuv,
           sem):
        c = lax.axis_index("c")
        s = lax.axis_index("s")
        r0 = pl.multiple_of(s * rows_t, 8)
        col = pl.multiple_of(c * HC, HC)
        xaccs = [xa0, xa1, xa2]
        xcols = [x0_hbm, x1_hbm, x2_hbm]
        pltpu.sync_copy(h_hbm.at[pl.ds(r0, rows_t), pl.ds(col, HC)],
                        hacc.at[pl.ds(r0, rows_t), :])
        if ntail:
            @pl.when(s == 0)
            def _():
                pltpu.sync_copy(h_hbm.at[pl.ds(tail0, ntail), pl.ds(col, HC)],
                                hacc.at[pl.ds(tail0, ntail), :])
        @pl.when((s == 0) & (c == 0))
        def _():
            for q, src_ref in enumerate([i00, i01, i02]):
                pltpu.sync_copy(src_ref, xaccs[q])
        @pl.when((s == 0) & (c == 1))
        def _():
            for q, src_ref in enumerate([i10, i11, i12]):
                pltpu.sync_copy(src_ref, xaccs[q])
        plsc.subcore_barrier()

        hbase = s * (E // 16)
        def hchunk(j, _):
            off = pl.multiple_of(hbase + j * K, 8)
            pltpu.sync_copy(dst_hbm.at[pl.ds(off, K)], dstv)
            pltpu.sync_copy(m_hbm.at[pl.ds(off, K), pl.ds(col, HC)], mv)
            pltpu.sync_copy(mv, hacc.at[dstv], add=True)
            return _
        lax.fori_loop(0, nh, hchunk, 0)

        nj = (ncx - s + 15) // 16
        def xchunk(j, _):
            cidx = s + j * 16
            off = pl.multiple_of(c * (E // 2) + cidx * KX, 8)
            pltpu.sync_copy(src_hbm.at[pl.ds(off, KX)], srcv)
            pltpu.sync_copy(dst_hbm.at[pl.ds(off, KX)], dstxv)
            pltpu.sync_copy(cw_hbm.at[pl.ds(off, KX)], cwv)
            for q in range(3):
                pltpu.async_copy(xcols[q].at[srcv], xsv, sem).wait()
                pltpu.async_copy(xcols[q].at[dstxv], xdv, sem).wait()
                for g in range(KX // 16):
                    d16 = pl.ds(g * 16, 16)
                    cuv[d16] = cwv[d16] * (xsv[d16] - xdv[d16])
                pltpu.sync_copy(cuv, xaccs[q].at[dstxv], add=True)
            return _
        lax.fori_loop(0, nj, xchunk, 0)
        plsc.subcore_barrier()

        pltpu.sync_copy(hacc.at[pl.ds(r0, rows_t), :],
                        ho_hbm.at[pl.ds(r0, rows_t), pl.ds(col, HC)])
        if ntail:
            @pl.when(s == 0)
            def _():
                pltpu.sync_copy(hacc.at[pl.ds(tail0, ntail), :],
                                ho_hbm.at[pl.ds(tail0, ntail), pl.ds(col, HC)])
        @pl.when((s == 0) & (c == 0))
        def _():
            for q, dst_ref in enumerate([p00, p01, p02]):
                pltpu.sync_copy(xaccs[q], dst_ref)
        @pl.when((s == 0) & (c == 1))
        def _():
            for q, dst_ref in enumerate([p10, p11, p12]):
                pltpu.sync_copy(xaccs[q], dst_ref)

    return sk(h, x0, x1, x2, *pxs, m, cw, src, dst)


def _pick_block(E):
    for b in (256, 128, 64, 32, 16, 8):
        if E % b == 0:
            return b
    return 8


def kernel(h, x, edge_index, edge_dist, W1, b1, W2, b2, Wc1, bc1, Wc2, We1, be1, We2, be2):
    N, D = h.shape
    E = edge_dist.shape[0]
    H = W1.shape[1]
    ED = We2.shape[0]
    B = _pick_block(E)
    src = edge_index[0]
    dst = edge_index[1]

    bf16 = jnp.bfloat16
    Wab = jnp.concatenate([W1[:D], Wc1[:D]], axis=1).astype(bf16)
    Wbb = jnp.concatenate([W1[D:2 * D], Wc1[D:2 * D]], axis=1).astype(bf16)
    Web = jnp.concatenate([W1[2 * D:], Wc1[2 * D:]], axis=1).astype(bf16)
    bf = jnp.concatenate([b1, bc1])[None, :]
    b2r = b2[None, :]
    wc2r = Wc2[:, 0].astype(bf16)
    be1r = be1[None, :]
    be2r = be2[None, :]

    weights = (Wab, Wbb, Web, bf, W2.astype(bf16), b2r, wc2r,
               We1, be1r, We2, be2r)
    full = lambda r, c: pl.BlockSpec((r, c), lambda i: (0, 0))

    def edge_mlp(hs, hd, ed2):
        ES = hs.shape[0]
        return pl.pallas_call(
            functools.partial(_edge_mlp_body, H=H),
            grid=(ES // B,),
            in_specs=[
                pl.BlockSpec((B, D), lambda i: (i, 0)),
                pl.BlockSpec((B, D), lambda i: (i, 0)),
                pl.BlockSpec((B, 1), lambda i: (i, 0)),
                full(D, 2 * H), full(D, 2 * H), full(ED, 2 * H), full(1, 2 * H),
                full(H, D), full(1, D),
                pl.BlockSpec((H,), lambda i: (0,)),
                full(1, ED), full(1, ED), full(ED, ED), full(1, ED),
            ],
            out_specs=[
                pl.BlockSpec((B, D), lambda i: (i, 0)),
                pl.BlockSpec((B,), lambda i: (i,)),
            ],
            out_shape=[
                jax.ShapeDtypeStruct((ES, D), jnp.float32),
                jax.ShapeDtypeStruct((ES,), jnp.float32),
            ],
        )(hs, hd, ed2, *weights)

    S = 5 if E % (5 * 32 * 200) == 0 else 1
    ES = E // S
    ho = h
    hb = h
    pxs = [jnp.zeros((N,), jnp.float32)] * 6
    x0, x1, x2 = x[:, 0], x[:, 1], x[:, 2]
    for si in range(S):
        sl = slice(si * ES, (si + 1) * ES)
        hs, hd = _sc_gather_pair(hb, src[sl], dst[sl])
        m, cw = edge_mlp(hs, hd, edge_dist[sl, None])
        ho, *pxs = _sc_scatter(ho, x0, x1, x2, pxs, m, cw, src[sl], dst[sl])
    p00, p01, p02, p10, p11, p12 = pxs
    x_out = x + jnp.stack([p00 + p10, p01 + p11, p02 + p12], axis=1)
    return (ho, x_out)


# speedup vs baseline: 1.0817x; 1.0068x over previous
import functools

import jax
import jax.numpy as jnp
from jax import lax
from jax.experimental import pallas as pl
from jax.experimental.pallas import tpu as pltpu
from jax.experimental.pallas import tpu_sc as plsc


def _sc_gather_pair(h, src, dst):
    N, D = h.shape
    E = src.shape[0]
    NW = 32
    per_w = E // NW
    K = 200
    nch = per_w // K
    dt = h.dtype
    mesh = plsc.VectorSubcoreMesh(core_axis_name="c", subcore_axis_name="s")

    @functools.partial(
        pl.kernel, mesh=mesh,
        out_type=[jax.ShapeDtypeStruct((E, D), dt),
                  jax.ShapeDtypeStruct((E, D), dt)],
        scratch_types=[pltpu.VMEM((2 * per_w,), jnp.int32),
                       pltpu.VMEM((K, D), dt),
                       pltpu.VMEM((K, D), dt),
                       pltpu.SemaphoreType.DMA,
                       pltpu.SemaphoreType.DMA,
                       pltpu.SemaphoreType.DMA,
                       pltpu.SemaphoreType.DMA],
    )
    def gk(h_hbm, src_hbm, dst_hbm, hs_hbm, hd_hbm,
           idx_v, rows0, rows1, sg0, sg1, sw0, sw1):
        wid = lax.axis_index("s") * 2 + lax.axis_index("c")
        base = wid * per_w
        pltpu.sync_copy(src_hbm.at[pl.ds(base, per_w)],
                        idx_v.at[pl.ds(0, per_w)])
        pltpu.sync_copy(dst_hbm.at[pl.ds(base, per_w)],
                        idx_v.at[pl.ds(per_w, per_w)])
        rows = (rows0, rows1)
        sg = (sg0, sg1)
        sw = (sw0, sw1)
        chunks = []
        for t in range(2 * nch):
            out = hs_hbm if t < nch else hd_hbm
            ioff = t * K
            ooff = base + (t % nch) * K
            chunks.append((out, ioff, ooff))
        gathers = []
        writes = [None, None]
        for t, (out, ioff, ooff) in enumerate(chunks):
            b = t % 2
            if writes[b] is not None:
                writes[b].wait()
            g = pltpu.async_copy(h_hbm.at[idx_v.at[pl.ds(ioff, K)]],
                                 rows[b], sg[b])
            gathers.append((g, t))
            if t >= 1:
                gprev, tp = gathers[t - 1]
                gprev.wait()
                outp, _, ooffp = chunks[tp]
                bp = tp % 2
                writes[bp] = pltpu.async_copy(
                    rows[bp], outp.at[pl.ds(ooffp, K), :], sw[bp])
        glast, tl = gathers[-1]
        glast.wait()
        outl, _, ooffl = chunks[tl]
        pltpu.sync_copy(rows[tl % 2], outl.at[pl.ds(ooffl, K), :])
        if writes[(tl - 1) % 2] is not None:
            writes[(tl - 1) % 2].wait()

    return gk(h, src, dst)


def _edge_mlp_body(hs_ref, hd_ref, ed_ref,
                   Wab_ref, Wbb_ref, Web_ref, bf_ref,
                   W2_ref, b2_ref, wc2_ref,
                   We1_ref, be1_ref, We2_ref, be2_ref,
                   m_ref, cw_ref, *, H):
    f32 = jnp.float32
    bf16 = jnp.bfloat16
    pre = (jnp.dot(hs_ref[...].astype(bf16), Wab_ref[...], preferred_element_type=f32)
           + jnp.dot(hd_ref[...].astype(bf16), Wbb_ref[...], preferred_element_type=f32)
           + bf_ref[...])
    d = ed_ref[...]
    eh = d * We1_ref[...] + be1_ref[...]
    eh = eh * jax.nn.sigmoid(eh)
    ea = jnp.dot(eh, We2_ref[...], preferred_element_type=f32) + be2_ref[...]
    pre = pre + jnp.dot(ea.astype(bf16), Web_ref[...], preferred_element_type=f32)
    preb = pre.astype(bf16)
    act = preb * jax.nn.sigmoid(preb)
    a_node = act[:, :H]
    a_coord = act[:, H:]
    m_ref[...] = jnp.dot(a_node, W2_ref[...], preferred_element_type=f32) + b2_ref[...]
    cw_ref[...] = jnp.sum(a_coord * wc2_ref[...].astype(f32), axis=1)


def _sc_scatter(h, x0, x1, x2, pxs, m, cw, src, dst):
    N, D = h.shape
    E = src.shape[0]
    HC = D // 2
    K = 80
    nh = E // 16 // K
    KX = E // 32
    rows_t = (N // 16) & ~7
    tail0 = rows_t * 16
    ntail = N - tail0
    mesh = plsc.VectorSubcoreMesh(core_axis_name="c", subcore_axis_name="s")

    @functools.partial(
        pl.kernel, mesh=mesh,
        out_type=[jax.ShapeDtypeStruct((N, D), jnp.float32)]
                 + [jax.ShapeDtypeStruct((N,), jnp.float32)] * 6,
        scratch_types=[pltpu.VMEM_SHARED((N, HC), jnp.float32),
                       pltpu.VMEM_SHARED((N,), jnp.float32),
                       pltpu.VMEM_SHARED((N,), jnp.float32),
                       pltpu.VMEM_SHARED((N,), jnp.float32),
                       pltpu.VMEM((K,), jnp.int32),
                       pltpu.VMEM((K,), jnp.int32),
                       pltpu.VMEM((K, HC), jnp.float32),
                       pltpu.VMEM((K, HC), jnp.float32),
                       pltpu.VMEM((KX,), jnp.int32),
                       pltpu.VMEM((KX,), jnp.int32),
                       pltpu.VMEM((KX,), jnp.float32),
                       pltpu.VMEM((KX,), jnp.float32),
                       pltpu.VMEM((KX,), jnp.float32),
                       pltpu.VMEM((KX,), jnp.float32),
                       pltpu.VMEM((KX,), jnp.float32),
                       pltpu.VMEM((KX,), jnp.float32),
                       pltpu.VMEM((KX,), jnp.float32),
                       pltpu.VMEM((KX,), jnp.float32),
                       pltpu.VMEM((KX,), jnp.float32),
                       pltpu.VMEM((KX,), jnp.float32),
                       pltpu.SemaphoreType.DMA,
                       pltpu.SemaphoreType.DMA,
                       pltpu.SemaphoreType.DMA,
                       pltpu.SemaphoreType.DMA,
                       pltpu.SemaphoreType.DMA,
                       pltpu.SemaphoreType.DMA,
                       pltpu.SemaphoreType.DMA,
                       pltpu.SemaphoreType.DMA],
    )
    def sk(h_hbm, x0_hbm, x1_hbm, x2_hbm,
           i00, i01, i02, i10, i11, i12, m_hbm, cw_hbm,
           src_hbm, dst_hbm, ho_hbm, p00, p01, p02, p10, p11, p12,
           hacc, xa0, xa1, xa2, dst0, dst1, mv0, mv1,
           srcx, dstx, cwv, xs0, xs1, xs2, xd0, xd1, xd2, cu0, cu1, cu2,
           semA, semB, semI0, semI1, semM0, semM1, semS0, semS1):
        c = lax.axis_index("c")
        s = lax.axis_index("s")
        r0 = pl.multiple_of(s * rows_t, 8)
        col = pl.multiple_of(c * HC, HC)
        xaccs = [xa0, xa1, xa2]
        xcols = [x0_hbm, x1_hbm, x2_hbm]
        xss = [xs0, xs1, xs2]
        xds = [xd0, xd1, xd2]
        cus = [cu0, cu1, cu2]
        pltpu.sync_copy(h_hbm.at[pl.ds(r0, rows_t), pl.ds(col, HC)],
                        hacc.at[pl.ds(r0, rows_t), :])
        if ntail:
            @pl.when(s == 0)
            def _():
                pltpu.sync_copy(h_hbm.at[pl.ds(tail0, ntail), pl.ds(col, HC)],
                                hacc.at[pl.ds(tail0, ntail), :])
        @pl.when((s == 0) & (c == 0))
        def _():
            for q, src_ref in enumerate([i00, i01, i02]):
                pltpu.sync_copy(src_ref, xaccs[q])
        @pl.when((s == 0) & (c == 1))
        def _():
            for q, src_ref in enumerate([i10, i11, i12]):
                pltpu.sync_copy(src_ref, xaccs[q])
        plsc.subcore_barrier()

        xoff = pl.multiple_of(c * (E // 2) + s * KX, 8)
        la = pltpu.async_copy(src_hbm.at[pl.ds(xoff, KX)], srcx, semA)
        lb = pltpu.async_copy(dst_hbm.at[pl.ds(xoff, KX)], dstx, semA)
        lc = pltpu.async_copy(cw_hbm.at[pl.ds(xoff, KX)], cwv, semA)
        la.wait(); lb.wait(); lc.wait()
        gts = []
        for q in range(3):
            gts.append(pltpu.async_copy(xcols[q].at[srcx], xss[q], semB))
            gts.append(pltpu.async_copy(xcols[q].at[dstx], xds[q], semB))

        hbase = s * (E // 16)
        dsts = (dst0, dst1)
        mvs = (mv0, mv1)
        semIs = (semI0, semI1)
        semMs = (semM0, semM1)
        loads = [None, None]
        scat = [None, None]
        for j in range(nh):
            b = j % 2
            off = pl.multiple_of(hbase + j * K, 8)
            if scat[b] is not None:
                scat[b].wait()
            li = pltpu.async_copy(dst_hbm.at[pl.ds(off, K)], dsts[b], semIs[b])
            lm = pltpu.async_copy(m_hbm.at[pl.ds(off, K), pl.ds(col, HC)],
                                  mvs[b], semMs[b])
            loads[b] = (li, lm)
            li.wait(); lm.wait()
            scat[b] = pltpu.async_copy(mvs[b], hacc.at[dsts[b]],
                                       (semS0, semS1)[b], add=True)
        for b in range(2):
            if scat[b] is not None:
                scat[b].wait()

        for g in gts:
            g.wait()
        ngrp = KX // 16
        offs = [g * 16 for g in range(ngrp)]
        if KX % 16:
            offs.append(KX - 16)
        for q in range(3):
            for o in offs:
                d16 = pl.ds(o, 16)
                cus[q][d16] = cwv[d16] * (xss[q][d16] - xds[q][d16])
        for q in range(3):
            pltpu.sync_copy(cus[q], xaccs[q].at[dstx], add=True)
        plsc.subcore_barrier()

        pltpu.sync_copy(hacc.at[pl.ds(r0, rows_t), :],
                        ho_hbm.at[pl.ds(r0, rows_t), pl.ds(col, HC)])
        if ntail:
            @pl.when(s == 0)
            def _():
                pltpu.sync_copy(hacc.at[pl.ds(tail0, ntail), :],
                                ho_hbm.at[pl.ds(tail0, ntail), pl.ds(col, HC)])
        @pl.when((s == 0) & (c == 0))
        def _():
            for q, dst_ref in enumerate([p00, p01, p02]):
                pltpu.sync_copy(xaccs[q], dst_ref)
        @pl.when((s == 0) & (c == 1))
        def _():
            for q, dst_ref in enumerate([p10, p11, p12]):
                pltpu.sync_copy(xaccs[q], dst_ref)

    return sk(h, x0, x1, x2, *pxs, m, cw, src, dst)


def _pick_block(E):
    for b in (256, 128, 64, 32, 16, 8):
        if E % b == 0:
            return b
    return 8


def kernel(h, x, edge_index, edge_dist, W1, b1, W2, b2, Wc1, bc1, Wc2, We1, be1, We2, be2):
    N, D = h.shape
    E = edge_dist.shape[0]
    H = W1.shape[1]
    ED = We2.shape[0]
    B = _pick_block(E)
    src = edge_index[0]
    dst = edge_index[1]

    bf16 = jnp.bfloat16
    Wab = jnp.concatenate([W1[:D], Wc1[:D]], axis=1).astype(bf16)
    Wbb = jnp.concatenate([W1[D:2 * D], Wc1[D:2 * D]], axis=1).astype(bf16)
    Web = jnp.concatenate([W1[2 * D:], Wc1[2 * D:]], axis=1).astype(bf16)
    bf = jnp.concatenate([b1, bc1])[None, :]
    b2r = b2[None, :]
    wc2r = Wc2[:, 0].astype(bf16)
    be1r = be1[None, :]
    be2r = be2[None, :]

    weights = (Wab, Wbb, Web, bf, W2.astype(bf16), b2r, wc2r,
               We1, be1r, We2, be2r)
    full = lambda r, c: pl.BlockSpec((r, c), lambda i: (0, 0))

    def edge_mlp(hs, hd, ed2):
        ES = hs.shape[0]
        return pl.pallas_call(
            functools.partial(_edge_mlp_body, H=H),
            grid=(ES // B,),
            in_specs=[
                pl.BlockSpec((B, D), lambda i: (i, 0)),
                pl.BlockSpec((B, D), lambda i: (i, 0)),
                pl.BlockSpec((B, 1), lambda i: (i, 0)),
                full(D, 2 * H), full(D, 2 * H), full(ED, 2 * H), full(1, 2 * H),
                full(H, D), full(1, D),
                pl.BlockSpec((H,), lambda i: (0,)),
                full(1, ED), full(1, ED), full(ED, ED), full(1, ED),
            ],
            out_specs=[
                pl.BlockSpec((B, D), lambda i: (i, 0)),
                pl.BlockSpec((B,), lambda i: (i,)),
            ],
            out_shape=[
                jax.ShapeDtypeStruct((ES, D), jnp.float32),
                jax.ShapeDtypeStruct((ES,), jnp.float32),
            ],
        )(hs, hd, ed2, *weights)

    S = 5 if E % (5 * 32 * 200) == 0 else 1
    ES = E // S
    ho = h
    hb = h
    pxs = [jnp.zeros((N,), jnp.float32)] * 6
    x0, x1, x2 = x[:, 0], x[:, 1], x[:, 2]
    for si in range(S):
        sl = slice(si * ES, (si + 1) * ES)
        hs, hd = _sc_gather_pair(hb, src[sl], dst[sl])
        m, cw = edge_mlp(hs, hd, edge_dist[sl, None])
        ho, *pxs = _sc_scatter(ho, x0, x1, x2, pxs, m, cw, src[sl], dst[sl])
    p00, p01, p02, p10, p11, p12 = pxs
    x_out = x + jnp.stack([p00 + p10, p01 + p11, p02 + p12], axis=1)
    return (ho, x_out)


# speedup vs baseline: 1.0835x; 1.0017x over previous
import functools

import jax
import jax.numpy as jnp
from jax import lax
from jax.experimental import pallas as pl
from jax.experimental.pallas import tpu as pltpu
from jax.experimental.pallas import tpu_sc as plsc


def _sc_gather_pair(h, src, dst):
    N, D = h.shape
    E = src.shape[0]
    NW = 32
    per_w = E // NW
    K = 200
    nch = per_w // K
    dt = h.dtype
    mesh = plsc.VectorSubcoreMesh(core_axis_name="c", subcore_axis_name="s")

    @functools.partial(
        pl.kernel, mesh=mesh,
        out_type=[jax.ShapeDtypeStruct((E, D), dt),
                  jax.ShapeDtypeStruct((E, D), dt)],
        scratch_types=[pltpu.VMEM((2 * per_w,), jnp.int32),
                       pltpu.VMEM((K, D), dt),
                       pltpu.VMEM((K, D), dt),
                       pltpu.SemaphoreType.DMA,
                       pltpu.SemaphoreType.DMA,
                       pltpu.SemaphoreType.DMA,
                       pltpu.SemaphoreType.DMA],
    )
    def gk(h_hbm, src_hbm, dst_hbm, hs_hbm, hd_hbm,
           idx_v, rows0, rows1, sg0, sg1, sw0, sw1):
        wid = lax.axis_index("s") * 2 + lax.axis_index("c")
        base = wid * per_w
        pltpu.sync_copy(src_hbm.at[pl.ds(base, per_w)],
                        idx_v.at[pl.ds(0, per_w)])
        pltpu.sync_copy(dst_hbm.at[pl.ds(base, per_w)],
                        idx_v.at[pl.ds(per_w, per_w)])
        rows = (rows0, rows1)
        sg = (sg0, sg1)
        sw = (sw0, sw1)
        chunks = []
        for t in range(2 * nch):
            out = hs_hbm if t < nch else hd_hbm
            ioff = t * K
            ooff = base + (t % nch) * K
            chunks.append((out, ioff, ooff))
        gathers = []
        writes = [None, None]
        for t, (out, ioff, ooff) in enumerate(chunks):
            b = t % 2
            if writes[b] is not None:
                writes[b].wait()
            g = pltpu.async_copy(h_hbm.at[idx_v.at[pl.ds(ioff, K)]],
                                 rows[b], sg[b])
            gathers.append((g, t))
            if t >= 1:
                gprev, tp = gathers[t - 1]
                gprev.wait()
                outp, _, ooffp = chunks[tp]
                bp = tp % 2
                writes[bp] = pltpu.async_copy(
                    rows[bp], outp.at[pl.ds(ooffp, K), :], sw[bp])
        glast, tl = gathers[-1]
        glast.wait()
        outl, _, ooffl = chunks[tl]
        pltpu.sync_copy(rows[tl % 2], outl.at[pl.ds(ooffl, K), :])
        if writes[(tl - 1) % 2] is not None:
            writes[(tl - 1) % 2].wait()

    return gk(h, src, dst)


def _edge_mlp_body(hs_ref, hd_ref, ed_ref,
                   Wab_ref, Wbb_ref, Web_ref, bf_ref,
                   W2_ref, b2_ref, wc2_ref,
                   We1_ref, be1_ref, We2_ref, be2_ref,
                   m_ref, cw_ref, *, H):
    f32 = jnp.float32
    bf16 = jnp.bfloat16
    pre = (jnp.dot(hs_ref[...].astype(bf16), Wab_ref[...], preferred_element_type=f32)
           + jnp.dot(hd_ref[...].astype(bf16), Wbb_ref[...], preferred_element_type=f32)
           + bf_ref[...])
    d = ed_ref[...]
    eh = d * We1_ref[...] + be1_ref[...]
    eh = eh * jax.nn.sigmoid(eh)
    ea = jnp.dot(eh, We2_ref[...], preferred_element_type=f32) + be2_ref[...]
    pre = pre + jnp.dot(ea.astype(bf16), Web_ref[...], preferred_element_type=f32)
    preb = pre.astype(bf16)
    act = preb * jax.nn.sigmoid(preb)
    a_node = act[:, :H]
    a_coord = act[:, H:]
    m_ref[...] = jnp.dot(a_node, W2_ref[...], preferred_element_type=f32) + b2_ref[...]
    cw_ref[...] = jnp.sum(a_coord * wc2_ref[...], axis=1)


def _sc_scatter(h, x0, x1, x2, pxs, m, cw, src, dst):
    N, D = h.shape
    E = src.shape[0]
    HC = D // 2
    K = 80
    nh = E // 16 // K
    KX = E // 32
    rows_t = (N // 16) & ~7
    tail0 = rows_t * 16
    ntail = N - tail0
    mesh = plsc.VectorSubcoreMesh(core_axis_name="c", subcore_axis_name="s")

    @functools.partial(
        pl.kernel, mesh=mesh,
        out_type=[jax.ShapeDtypeStruct((N, D), jnp.float32)]
                 + [jax.ShapeDtypeStruct((N,), jnp.float32)] * 6,
        scratch_types=[pltpu.VMEM_SHARED((N, HC), jnp.float32),
                       pltpu.VMEM_SHARED((N,), jnp.float32),
                       pltpu.VMEM_SHARED((N,), jnp.float32),
                       pltpu.VMEM_SHARED((N,), jnp.float32),
                       pltpu.VMEM((K,), jnp.int32),
                       pltpu.VMEM((K,), jnp.int32),
                       pltpu.VMEM((K, HC), jnp.float32),
                       pltpu.VMEM((K, HC), jnp.float32),
                       pltpu.VMEM((KX,), jnp.int32),
                       pltpu.VMEM((KX,), jnp.int32),
                       pltpu.VMEM((KX,), jnp.float32),
                       pltpu.VMEM((KX,), jnp.float32),
                       pltpu.VMEM((KX,), jnp.float32),
                       pltpu.VMEM((KX,), jnp.float32),
                       pltpu.VMEM((KX,), jnp.float32),
                       pltpu.VMEM((KX,), jnp.float32),
                       pltpu.VMEM((KX,), jnp.float32),
                       pltpu.VMEM((KX,), jnp.float32),
                       pltpu.VMEM((KX,), jnp.float32),
                       pltpu.VMEM((KX,), jnp.float32),
                       pltpu.SemaphoreType.DMA,
                       pltpu.SemaphoreType.DMA,
                       pltpu.SemaphoreType.DMA,
                       pltpu.SemaphoreType.DMA,
                       pltpu.SemaphoreType.DMA,
                       pltpu.SemaphoreType.DMA,
                       pltpu.SemaphoreType.DMA,
                       pltpu.SemaphoreType.DMA],
    )
    def sk(h_hbm, x0_hbm, x1_hbm, x2_hbm,
           i00, i01, i02, i10, i11, i12, m_hbm, cw_hbm,
           src_hbm, dst_hbm, ho_hbm, p00, p01, p02, p10, p11, p12,
           hacc, xa0, xa1, xa2, dst0, dst1, mv0, mv1,
           srcx, dstx, cwv, xs0, xs1, xs2, xd0, xd1, xd2, cu0, cu1, cu2,
           semA, semB, semI0, semI1, semM0, semM1, semS0, semS1):
        c = lax.axis_index("c")
        s = lax.axis_index("s")
        r0 = pl.multiple_of(s * rows_t, 8)
        col = pl.multiple_of(c * HC, HC)
        xaccs = [xa0, xa1, xa2]
        xcols = [x0_hbm, x1_hbm, x2_hbm]
        xss = [xs0, xs1, xs2]
        xds = [xd0, xd1, xd2]
        cus = [cu0, cu1, cu2]
        pltpu.sync_copy(h_hbm.at[pl.ds(r0, rows_t), pl.ds(col, HC)],
                        hacc.at[pl.ds(r0, rows_t), :])
        if ntail:
            @pl.when(s == 0)
            def _():
                pltpu.sync_copy(h_hbm.at[pl.ds(tail0, ntail), pl.ds(col, HC)],
                                hacc.at[pl.ds(tail0, ntail), :])
        @pl.when((s == 0) & (c == 0))
        def _():
            for q, src_ref in enumerate([i00, i01, i02]):
                pltpu.sync_copy(src_ref, xaccs[q])
        @pl.when((s == 0) & (c == 1))
        def _():
            for q, src_ref in enumerate([i10, i11, i12]):
                pltpu.sync_copy(src_ref, xaccs[q])
        plsc.subcore_barrier()

        xoff = pl.multiple_of(c * (E // 2) + s * KX, 8)
        la = pltpu.async_copy(src_hbm.at[pl.ds(xoff, KX)], srcx, semA)
        lb = pltpu.async_copy(dst_hbm.at[pl.ds(xoff, KX)], dstx, semA)
        lc = pltpu.async_copy(cw_hbm.at[pl.ds(xoff, KX)], cwv, semA)
        la.wait(); lb.wait(); lc.wait()
        gts = []
        for q in range(3):
            gts.append(pltpu.async_copy(xcols[q].at[srcx], xss[q], semB))
            gts.append(pltpu.async_copy(xcols[q].at[dstx], xds[q], semB))

        hbase = s * (E // 16)
        dsts = (dst0, dst1)
        mvs = (mv0, mv1)
        semIs = (semI0, semI1)
        semMs = (semM0, semM1)
        loads = [None, None]
        scat = [None, None]
        for j in range(nh):
            b = j % 2
            off = pl.multiple_of(hbase + j * K, 8)
            if scat[b] is not None:
                scat[b].wait()
            li = pltpu.async_copy(dst_hbm.at[pl.ds(off, K)], dsts[b], semIs[b])
            lm = pltpu.async_copy(m_hbm.at[pl.ds(off, K), pl.ds(col, HC)],
                                  mvs[b], semMs[b])
            loads[b] = (li, lm)
            li.wait(); lm.wait()
            scat[b] = pltpu.async_copy(mvs[b], hacc.at[dsts[b]],
                                       (semS0, semS1)[b], add=True)
        for b in range(2):
            if scat[b] is not None:
                scat[b].wait()

        for g in gts:
            g.wait()
        ngrp = KX // 16
        offs = [g * 16 for g in range(ngrp)]
        if KX % 16:
            offs.append(KX - 16)
        for q in range(3):
            for o in offs:
                d16 = pl.ds(o, 16)
                cus[q][d16] = cwv[d16] * (xss[q][d16] - xds[q][d16])
        for q in range(3):
            pltpu.sync_copy(cus[q], xaccs[q].at[dstx], add=True)
        plsc.subcore_barrier()

        pltpu.sync_copy(hacc.at[pl.ds(r0, rows_t), :],
                        ho_hbm.at[pl.ds(r0, rows_t), pl.ds(col, HC)])
        if ntail:
            @pl.when(s == 0)
            def _():
                pltpu.sync_copy(hacc.at[pl.ds(tail0, ntail), :],
                                ho_hbm.at[pl.ds(tail0, ntail), pl.ds(col, HC)])
        @pl.when((s == 0) & (c == 0))
        def _():
            for q, dst_ref in enumerate([p00, p01, p02]):
                pltpu.sync_copy(xaccs[q], dst_ref)
        @pl.when((s == 0) & (c == 1))
        def _():
            for q, dst_ref in enumerate([p10, p11, p12]):
                pltpu.sync_copy(xaccs[q], dst_ref)

    return sk(h, x0, x1, x2, *pxs, m, cw, src, dst)


def _pick_block(E):
    for b in (256, 128, 64, 32, 16, 8):
        if E % b == 0:
            return b
    return 8


def kernel(h, x, edge_index, edge_dist, W1, b1, W2, b2, Wc1, bc1, Wc2, We1, be1, We2, be2):
    N, D = h.shape
    E = edge_dist.shape[0]
    H = W1.shape[1]
    ED = We2.shape[0]
    B = _pick_block(E)
    src = edge_index[0]
    dst = edge_index[1]

    bf16 = jnp.bfloat16
    Wab = jnp.concatenate([W1[:D], Wc1[:D]], axis=1).astype(bf16)
    Wbb = jnp.concatenate([W1[D:2 * D], Wc1[D:2 * D]], axis=1).astype(bf16)
    Web = jnp.concatenate([W1[2 * D:], Wc1[2 * D:]], axis=1).astype(bf16)
    bf = jnp.concatenate([b1, bc1])[None, :]
    b2r = b2[None, :]
    wc2r = Wc2[:, 0][None, :]
    be1r = be1[None, :]
    be2r = be2[None, :]

    weights = (Wab, Wbb, Web, bf, W2.astype(bf16), b2r, wc2r,
               We1, be1r, We2, be2r)
    full = lambda r, c: pl.BlockSpec((r, c), lambda i: (0, 0))

    def edge_mlp(hs, hd, ed2):
        ES = hs.shape[0]
        return pl.pallas_call(
            functools.partial(_edge_mlp_body, H=H),
            grid=(ES // B,),
            in_specs=[
                pl.BlockSpec((B, D), lambda i: (i, 0)),
                pl.BlockSpec((B, D), lambda i: (i, 0)),
                pl.BlockSpec((B, 1), lambda i: (i, 0)),
                full(D, 2 * H), full(D, 2 * H), full(ED, 2 * H), full(1, 2 * H),
                full(H, D), full(1, D), full(1, H),
                full(1, ED), full(1, ED), full(ED, ED), full(1, ED),
            ],
            out_specs=[
                pl.BlockSpec((B, D), lambda i: (i, 0)),
                pl.BlockSpec((B,), lambda i: (i,)),
            ],
            out_shape=[
                jax.ShapeDtypeStruct((ES, D), jnp.float32),
                jax.ShapeDtypeStruct((ES,), jnp.float32),
            ],
        )(hs, hd, ed2, *weights)

    S = 5 if E % (5 * 32 * 200) == 0 else 1
    ES = E // S
    ho = h
    hb = h
    pxs = [jnp.zeros((N,), jnp.float32)] * 6
    x0, x1, x2 = x[:, 0], x[:, 1], x[:, 2]
    for si in range(S):
        sl = slice(si * ES, (si + 1) * ES)
        hs, hd = _sc_gather_pair(hb, src[sl], dst[sl])
        m, cw = edge_mlp(hs, hd, edge_dist[sl, None])
        ho, *pxs = _sc_scatter(ho, x0, x1, x2, pxs, m, cw, src[sl], dst[sl])
    p00, p01, p02, p10, p11, p12 = pxs
    x_out = x + jnp.stack([p00 + p10, p01 + p11, p02 + p12], axis=1)
    return (ho, x_out)
